# Initial kernel scaffold; baseline (speedup 1.0000x reference)
import jax
import jax.numpy as jnp
from jax.experimental import pallas as pl


def kernel(font_output, font_target, use_hard_mining):
    raise NotImplementedError("write your pallas kernel here")


# speedup vs baseline: 36.4167x; 36.4167x over previous
import functools

import jax
import jax.numpy as jnp
from jax import lax
from jax.experimental import pallas as pl
from jax.experimental.pallas import tpu as pltpu
from jax.experimental.pallas import tpu_sc as plsc

_L = 16
_K = 25
_NEG_INF = float("-inf")
_CHUNK = 8192
_SUBS = 16
_VPS = _CHUNK // (_SUBS * _L)
_CAND = 4096
_OUTW = 32


@functools.cache
def _sc_collect(n_elems):
    info = plsc.get_sparse_core_info()
    nc, ns = info.num_cores, info.num_subcores
    nw = nc * ns
    slice_len = n_elems // nw
    chunks = slice_len // _CHUNK
    assert slice_len % _CHUNK == 0

    mesh = plsc.VectorSubcoreMesh(core_axis_name="c", subcore_axis_name="s")

    @functools.partial(
        pl.kernel,
        out_type=(
            jax.ShapeDtypeStruct((nw * _OUTW,), jnp.float32),
            jax.ShapeDtypeStruct((nw * _OUTW,), jnp.float32),
        ),
        mesh=mesh,
        compiler_params=pltpu.CompilerParams(needs_layout_passes=False),
        scratch_types=[
            pltpu.VMEM((_CHUNK,), jnp.float32),
            pltpu.VMEM((_CHUNK,), jnp.float32),
            pltpu.VMEM((_CAND,), jnp.float32),
            pltpu.VMEM((_CAND,), jnp.float32),
            pltpu.VMEM((_OUTW,), jnp.float32),
            pltpu.VMEM((_OUTW,), jnp.float32),
        ],
    )
    def collect(x_hbm, t_hbm, outn_hbm, outp_hbm, xb, tb, cna, cnb, oa, ob):
        ninf = jnp.full((_L,), _NEG_INF, jnp.float32)
        zc = jnp.zeros((_L,), jnp.int32)
        iot = lax.iota(jnp.int32, _L)
        wid = lax.axis_index("s") * nc + lax.axis_index("c")
        base = wid * slice_len

        def initb(j, _):
            cna[pl.ds(j * _L, _L)] = ninf
            cnb[pl.ds(j * _L, _L)] = ninf
            return 0

        lax.fori_loop(0, _CAND // _L, initb, 0)

        def chunk_body(c, carry):
            (cnta, cntb, taua, taub, m1a, m2a, m1b, m2b) = carry
            off = base + c * _CHUNK
            pltpu.sync_copy(x_hbm.at[pl.ds(off, _CHUNK)], xb)
            pltpu.sync_copy(t_hbm.at[pl.ds(off, _CHUNK)], tb)

            def sub_body(s, scarry):
                (cnta, cntb, taua, taub, m1a, m2a, m1b, m2b) = scarry

                def vbody(i, vc):
                    (cnta, cntb, m1a, m2a, m1b, m2b) = vc
                    idx = (s * _VPS + i) * _L
                    xv = xb[pl.ds(idx, _L)]
                    tv = tb[pl.ds(idx, _L)]
                    isneg = tv < 0.5
                    a = jnp.where(isneg, xv, ninf)
                    b = jnp.where(isneg, ninf, -xv)
                    m2a = jnp.maximum(m2a, jnp.minimum(m1a, a))
                    m1a = jnp.maximum(m1a, a)
                    m2b = jnp.maximum(m2b, jnp.minimum(m1b, b))
                    m1b = jnp.maximum(m1b, b)
                    ka = a > taua
                    kb = b > taub
                    sa, _ = plsc.sort_key_val(
                        jnp.where(ka, a, ninf), a, descending=True)
                    sb, _ = plsc.sort_key_val(
                        jnp.where(kb, b, ninf), b, descending=True)
                    offa = jnp.minimum(cnta[0], _CAND - _L)
                    offb = jnp.minimum(cntb[0], _CAND - _L)
                    cna[pl.ds(offa, _L)] = sa
                    cnb[pl.ds(offb, _L)] = sb
                    cnta = cnta + plsc.all_reduce_population_count(ka)
                    cntb = cntb + plsc.all_reduce_population_count(kb)
                    return (cnta, cntb, m1a, m2a, m1b, m2b)

                (cnta, cntb, m1a, m2a, m1b, m2b) = lax.fori_loop(
                    0, _VPS, vbody, (cnta, cntb, m1a, m2a, m1b, m2b))
                taua = jnp.min(m2a)
                taub = jnp.min(m2b)
                return (cnta, cntb, taua, taub, m1a, m2a, m1b, m2b)

            return lax.fori_loop(
                0, _SUBS, sub_body,
                (cnta, cntb, taua, taub, m1a, m2a, m1b, m2b))

        carry = lax.fori_loop(
            0, chunks, chunk_body,
            (zc, zc, jnp.float32(_NEG_INF), jnp.float32(_NEG_INF),
             ninf, ninf, ninf, ninf))
        cnta, cntb = carry[0], carry[1]

        def select25(cref, cntv):
            nv = (jnp.minimum(cntv[0], _CAND) + (_L - 1)) // _L

            def kbody(k, kc):
                filled, o0, o1 = kc

                def smax(j, m):
                    return jnp.maximum(m, cref[pl.ds(j * _L, _L)])

                v = jnp.max(lax.fori_loop(0, nv, smax, ninf))

                def srm(j, cacc):
                    vr = cref[pl.ds(j * _L, _L)]
                    eq = vr == v
                    cref[pl.ds(j * _L, _L)] = jnp.where(eq, ninf, vr)
                    return cacc + plsc.all_reduce_population_count(eq)

                cvec = lax.fori_loop(0, nv, srm, zc)
                take = jnp.minimum(cvec[0], _K - filled)
                lo = filled
                hi = filled + take
                o0 = jnp.where((iot >= lo) & (iot < hi), v, o0)
                o1 = jnp.where(((iot + _L) >= lo) & ((iot + _L) < hi), v, o1)
                return (filled + take, o0, o1)

            _, o0, o1 = lax.fori_loop(0, _K, kbody, (jnp.int32(0), ninf, ninf))
            return o0, o1

        o0, o1 = select25(cna, cnta)
        oa[pl.ds(0, _L)] = o0
        oa[pl.ds(_L, _L)] = o1
        p0, p1 = select25(cnb, cntb)
        ob[pl.ds(0, _L)] = p0
        ob[pl.ds(_L, _L)] = p1
        pltpu.sync_copy(oa, outn_hbm.at[pl.ds(wid * _OUTW, _OUTW)])
        pltpu.sync_copy(ob, outp_hbm.at[pl.ds(wid * _OUTW, _OUTW)])

    return collect


def _merge_body(nref, pref, oref):
    lane = lax.broadcasted_iota(jnp.int32, (1, 128), 1)

    def select25_tc(arr0):
        def kbody(k, kc):
            filled, out, arr = kc
            v = jnp.max(arr)
            eq = arr == v
            c = jnp.sum(eq.astype(jnp.int32))
            take = jnp.minimum(c, _K - filled)
            out = jnp.where((lane >= filled) & (lane < filled + take), v, out)
            arr = jnp.where(eq, _NEG_INF, arr)
            return (filled + take, out, arr)

        _, out, _ = lax.fori_loop(
            0, _K, kbody,
            (jnp.int32(0), jnp.full((1, 128), _NEG_INF, jnp.float32), arr0))
        return out

    m25 = lane < _K
    seln = select25_tc(nref[...])
    selp = select25_tc(pref[...])
    pn = jax.nn.sigmoid(seln)
    tn = jnp.maximum(jnp.log(1.0 - pn), -100.0)
    neg_loss = -0.5 * jnp.sum(jnp.where(m25, tn, 0.0)) / _K
    pp = jax.nn.sigmoid(-selp)
    tp = jnp.maximum(jnp.log(pp), -100.0)
    pos_loss = -0.5 * jnp.sum(jnp.where(m25, tp, 0.0)) / _K
    oref[...] = jnp.where(lane == 0, pos_loss,
                          jnp.where(lane == 1, neg_loss, 0.0))


def kernel(font_output, font_target, use_hard_mining):
    x = font_output.reshape(-1)
    t = font_target.reshape(-1)
    n = x.shape[0]

    def hard(_):
        negc, posc = _sc_collect(n)(x, t)
        out = pl.pallas_call(
            _merge_body,
            out_shape=jax.ShapeDtypeStruct((1, 128), jnp.float32),
        )(negc.reshape(8, -1), posc.reshape(8, -1))
        return out[0, 0], out[0, 1]

    def soft(_):
        p = jax.nn.sigmoid(x)
        pos_mask = t == 1
        neg_mask = t == 0
        logp = jnp.clip(jnp.log(p), -100.0, None)
        log1mp = jnp.clip(jnp.log(1.0 - p), -100.0, None)
        pos_loss = 0.5 * jnp.sum(jnp.where(pos_mask, -logp, 0.0)) / jnp.sum(pos_mask)
        neg_loss = 0.5 * jnp.sum(jnp.where(neg_mask, -log1mp, 0.0)) / jnp.sum(neg_mask)
        return pos_loss, neg_loss

    pos_loss, neg_loss = lax.cond(use_hard_mining != 0, hard, soft, operand=None)
    return (pos_loss + neg_loss, pos_loss, neg_loss)
